# baseline (device time: 43211 ns/iter reference)
import jax
import jax.numpy as jnp
from jax import lax
from jax.experimental import pallas as pl
from jax.experimental.pallas import tpu as pltpu

T = 1024
D = 1024
F = 2048
E_LOCAL = 2
C = 288
H = C // 2


def kernel(x, assign, W1, W2):
    assign2d = assign.reshape(T, 1)

    def body(
        x_ref,
        assign_ref,
        w1_ref,
        w2_ref,
        out_ref,
        xsend_ref,
        xrecv_ref,
        rown_ref,
        rsend_ref,
        rrecv_ref,
        s1_ref,
        s2_ref,
        send_sems,
        recv_sems,
        dma_sems,
    ):
        peer = (
            lax.axis_index("x"),
            1 - lax.axis_index("y"),
            lax.axis_index("z"),
        )

        cp1 = pltpu.make_async_copy(w1_ref.at[0], s1_ref, dma_sems.at[0])
        cp2 = pltpu.make_async_copy(w2_ref.at[0], s2_ref, dma_sems.at[1])
        cp1.start()
        cp2.start()

        barrier = pltpu.get_barrier_semaphore()
        pl.semaphore_signal(
            barrier, inc=1, device_id=peer, device_id_type=pl.DeviceIdType.MESH
        )
        pl.semaphore_wait(barrier, 1)

        xb = x_ref[...].astype(jnp.bfloat16)

        my_y = lax.axis_index("y")
        i4 = lax.broadcasted_iota(jnp.int32, (T, 4), 1)
        e_col = jnp.where(i4 < 2, 2 * my_y + i4, 2 * (1 - my_y) + i4 - 2)
        onehot = (assign_ref[...] == e_col).astype(jnp.bfloat16)
        tril = (
            lax.broadcasted_iota(jnp.int32, (T, T), 0)
            >= lax.broadcasted_iota(jnp.int32, (T, T), 1)
        ).astype(jnp.bfloat16)
        pos = (
            jnp.dot(tril, onehot, preferred_element_type=jnp.float32) - 1.0
        ).astype(jnp.int32)
        iota_c = lax.broadcasted_iota(jnp.int32, (T, C), 1)

        def disp_block(b):
            return (pos[:, b : b + 1] == iota_c).astype(
                jnp.bfloat16
            ) * onehot[:, b : b + 1]

        def dispatch(db):
            return lax.dot_general(
                db,
                xb,
                (((0,), (0,)), ((), ())),
                preferred_element_type=jnp.float32,
            ).astype(jnp.bfloat16)

        def remote(src, dst, k):
            return pltpu.make_async_remote_copy(
                src_ref=src,
                dst_ref=dst,
                send_sem=send_sems.at[k],
                recv_sem=recv_sems.at[k],
                device_id=peer,
                device_id_type=pl.DeviceIdType.MESH,
            )

        rdma_x = []
        for j in range(E_LOCAL):
            xsend_ref[pl.ds(j * C, C), :] = dispatch(disp_block(2 + j))
            rdma = remote(
                xsend_ref.at[pl.ds(j * C, C)], xrecv_ref.at[pl.ds(j * C, C)], j
            )
            rdma.start()
            rdma_x.append(rdma)

        d_own = []
        rdma_r = []
        for j in range(E_LOCAL):
            cp1.wait()
            cp2.wait()
            w1bj = s1_ref[...].astype(jnp.bfloat16)
            w2bj = s2_ref[...].astype(jnp.bfloat16)
            if j + 1 < E_LOCAL:
                cp1 = pltpu.make_async_copy(
                    w1_ref.at[j + 1], s1_ref, dma_sems.at[2]
                )
                cp2 = pltpu.make_async_copy(
                    w2_ref.at[j + 1], s2_ref, dma_sems.at[3]
                )
                cp1.start()
                cp2.start()

            dj = disp_block(j)
            d_own.append(dj)
            rdma_x[j].wait()
            inp = jnp.concatenate(
                [dispatch(dj), xrecv_ref[pl.ds(j * C, C), :]], axis=0
            )
            h = jnp.maximum(
                jnp.dot(inp, w1bj, preferred_element_type=jnp.float32), 0.0
            ).astype(jnp.bfloat16)
            rown_ref[pl.ds(j * C, C), :] = jnp.dot(
                h[:C], w2bj, preferred_element_type=jnp.float32
            ).astype(jnp.bfloat16)
            for k in range(2):
                lo = j * C + k * H
                rsend_ref[pl.ds(lo, H), :] = jnp.dot(
                    h[C + k * H : C + (k + 1) * H],
                    w2bj,
                    preferred_element_type=jnp.float32,
                ).astype(jnp.bfloat16)
                rdma = remote(
                    rsend_ref.at[pl.ds(lo, H)],
                    rrecv_ref.at[pl.ds(lo, H)],
                    2 + 2 * j + k,
                )
                rdma.start()
                rdma_r.append(rdma)

        def combine(db, res):
            return lax.dot_general(
                db,
                res,
                (((1,), (0,)), ((), ())),
                preferred_element_type=jnp.float32,
            )

        acc = combine(d_own[0], rown_ref[pl.ds(0, C), :]) + combine(
            d_own[1], rown_ref[pl.ds(C, C), :]
        )
        d_peer = [disp_block(2), disp_block(3)]
        for i in range(4):
            j, k = divmod(i, 2)
            rdma_r[i].wait()
            acc = acc + combine(
                d_peer[j][:, k * H : (k + 1) * H],
                rrecv_ref[pl.ds(j * C + k * H, H), :],
            )
        out_ref[...] = acc.astype(jnp.bfloat16)

    return pl.pallas_call(
        body,
        out_shape=jax.ShapeDtypeStruct((T, D), jnp.bfloat16),
        in_specs=[
            pl.BlockSpec(memory_space=pltpu.VMEM),
            pl.BlockSpec(memory_space=pltpu.VMEM),
            pl.BlockSpec(memory_space=pl.ANY),
            pl.BlockSpec(memory_space=pl.ANY),
        ],
        out_specs=pl.BlockSpec(memory_space=pltpu.VMEM),
        scratch_shapes=[
            pltpu.VMEM((2 * C, D), jnp.bfloat16),
            pltpu.VMEM((2 * C, D), jnp.bfloat16),
            pltpu.VMEM((2 * C, D), jnp.bfloat16),
            pltpu.VMEM((2 * C, D), jnp.bfloat16),
            pltpu.VMEM((2 * C, D), jnp.bfloat16),
            pltpu.VMEM((D, F), jnp.float32),
            pltpu.VMEM((F, D), jnp.float32),
            pltpu.SemaphoreType.DMA((6,)),
            pltpu.SemaphoreType.DMA((6,)),
            pltpu.SemaphoreType.DMA((4,)),
        ],
        compiler_params=pltpu.CompilerParams(
            collective_id=0, vmem_limit_bytes=100 * 1024 * 1024
        ),
    )(x, assign2d, W1, W2)


# device time: 41162 ns/iter; 1.0498x vs baseline; 1.0498x over previous
import jax
import jax.numpy as jnp
from jax import lax
from jax.experimental import pallas as pl
from jax.experimental.pallas import tpu as pltpu

T = 1024
D = 1024
F = 2048
E_LOCAL = 2
C = 288
H = C // 2


def kernel(x, assign, W1, W2):
    assign2d = assign.reshape(T, 1)

    def body(
        x_ref,
        assign_ref,
        w1_ref,
        w2_ref,
        out_ref,
        xsend_ref,
        xrecv_ref,
        rown_ref,
        rsend_ref,
        rrecv_ref,
        s1_ref,
        s2_ref,
        send_sems,
        recv_sems,
        dma_sems,
    ):
        peer = (
            lax.axis_index("x"),
            1 - lax.axis_index("y"),
            lax.axis_index("z"),
        )

        cp1 = pltpu.make_async_copy(w1_ref.at[0], s1_ref, dma_sems.at[0])
        cp2 = pltpu.make_async_copy(w2_ref.at[0], s2_ref, dma_sems.at[1])
        cp1.start()
        cp2.start()

        barrier = pltpu.get_barrier_semaphore()
        pl.semaphore_signal(
            barrier, inc=1, device_id=peer, device_id_type=pl.DeviceIdType.MESH
        )
        pl.semaphore_wait(barrier, 1)

        xb = x_ref[...].astype(jnp.bfloat16)

        my_y = lax.axis_index("y")
        i4 = lax.broadcasted_iota(jnp.int32, (T, 4), 1)
        e_col = jnp.where(i4 < 2, 2 * my_y + i4, 2 * (1 - my_y) + i4 - 2)
        onehot = (assign_ref[...] == e_col).astype(jnp.bfloat16)
        tril = (
            lax.broadcasted_iota(jnp.int32, (T, T), 0)
            >= lax.broadcasted_iota(jnp.int32, (T, T), 1)
        ).astype(jnp.bfloat16)
        pos = (
            jnp.dot(tril, onehot, preferred_element_type=jnp.float32) - 1.0
        ).astype(jnp.int32)
        iota_c = lax.broadcasted_iota(jnp.int32, (T, C), 1)

        def disp_block(b):
            return (pos[:, b : b + 1] == iota_c).astype(
                jnp.bfloat16
            ) * onehot[:, b : b + 1]

        def dispatch(db):
            return lax.dot_general(
                db,
                xb,
                (((0,), (0,)), ((), ())),
                preferred_element_type=jnp.float32,
            ).astype(jnp.bfloat16)

        def remote(src, dst, k):
            return pltpu.make_async_remote_copy(
                src_ref=src,
                dst_ref=dst,
                send_sem=send_sems.at[k],
                recv_sem=recv_sems.at[k],
                device_id=peer,
                device_id_type=pl.DeviceIdType.MESH,
            )

        rdma_x = []
        for j in range(E_LOCAL):
            xsend_ref[pl.ds(j * C, C), :] = dispatch(disp_block(2 + j))
            rdma = remote(
                xsend_ref.at[pl.ds(j * C, C)], xrecv_ref.at[pl.ds(j * C, C)], j
            )
            rdma.start()
            rdma_x.append(rdma)

        def ffn(inp, w1b, w2b):
            h = jnp.maximum(
                jnp.dot(inp, w1b, preferred_element_type=jnp.float32), 0.0
            ).astype(jnp.bfloat16)
            return jnp.dot(h, w2b, preferred_element_type=jnp.float32)

        d_own = []
        rdma_r = []
        for j in range(E_LOCAL):
            cp1.wait()
            cp2.wait()
            w1bj = s1_ref[...].astype(jnp.bfloat16)
            w2bj = s2_ref[...].astype(jnp.bfloat16)
            if j + 1 < E_LOCAL:
                cp1 = pltpu.make_async_copy(
                    w1_ref.at[j + 1], s1_ref, dma_sems.at[2]
                )
                cp2 = pltpu.make_async_copy(
                    w2_ref.at[j + 1], s2_ref, dma_sems.at[3]
                )
                cp1.start()
                cp2.start()

            dj = disp_block(j)
            d_own.append(dj)
            rown_ref[pl.ds(j * C, C), :] = ffn(
                dispatch(dj), w1bj, w2bj
            ).astype(jnp.bfloat16)

            rdma_x[j].wait()
            for k in range(2):
                lo = j * C + k * H
                rsend_ref[pl.ds(lo, H), :] = ffn(
                    xrecv_ref[pl.ds(lo, H), :], w1bj, w2bj
                ).astype(jnp.bfloat16)
                rdma = remote(
                    rsend_ref.at[pl.ds(lo, H)],
                    rrecv_ref.at[pl.ds(lo, H)],
                    2 + 2 * j + k,
                )
                rdma.start()
                rdma_r.append(rdma)

        def combine(db, res):
            return lax.dot_general(
                db,
                res,
                (((1,), (0,)), ((), ())),
                preferred_element_type=jnp.float32,
            )

        acc = combine(d_own[0], rown_ref[pl.ds(0, C), :]) + combine(
            d_own[1], rown_ref[pl.ds(C, C), :]
        )
        d_peer = [disp_block(2), disp_block(3)]
        for i in range(4):
            j, k = divmod(i, 2)
            rdma_r[i].wait()
            acc = acc + combine(
                d_peer[j][:, k * H : (k + 1) * H],
                rrecv_ref[pl.ds(j * C + k * H, H), :],
            )
        out_ref[...] = acc.astype(jnp.bfloat16)

    return pl.pallas_call(
        body,
        out_shape=jax.ShapeDtypeStruct((T, D), jnp.bfloat16),
        in_specs=[
            pl.BlockSpec(memory_space=pltpu.VMEM),
            pl.BlockSpec(memory_space=pltpu.VMEM),
            pl.BlockSpec(memory_space=pl.ANY),
            pl.BlockSpec(memory_space=pl.ANY),
        ],
        out_specs=pl.BlockSpec(memory_space=pltpu.VMEM),
        scratch_shapes=[
            pltpu.VMEM((2 * C, D), jnp.bfloat16),
            pltpu.VMEM((2 * C, D), jnp.bfloat16),
            pltpu.VMEM((2 * C, D), jnp.bfloat16),
            pltpu.VMEM((2 * C, D), jnp.bfloat16),
            pltpu.VMEM((2 * C, D), jnp.bfloat16),
            pltpu.VMEM((D, F), jnp.float32),
            pltpu.VMEM((F, D), jnp.float32),
            pltpu.SemaphoreType.DMA((6,)),
            pltpu.SemaphoreType.DMA((6,)),
            pltpu.SemaphoreType.DMA((4,)),
        ],
        compiler_params=pltpu.CompilerParams(
            collective_id=0, vmem_limit_bytes=100 * 1024 * 1024
        ),
    )(x, assign2d, W1, W2)
